# Initial kernel scaffold; baseline (speedup 1.0000x reference)
import jax
import jax.numpy as jnp
from jax.experimental import pallas as pl


def kernel(inputs, targets, smooth):
    raise NotImplementedError("write your pallas kernel here")


# speedup vs baseline: 34.5641x; 34.5641x over previous
import functools

import jax
import jax.numpy as jnp
from jax import lax
from jax.experimental import pallas as pl
from jax.experimental.pallas import tpu as pltpu
from jax.experimental.pallas import tpu_sc as plsc

_NC = 2
_NS = 16
_NW = _NC * _NS
_L = 16

_N_TOTAL = 64 * 512 * 512
_PER_W = _N_TOTAL // _NW
_CHUNK = 16384
_N_CHUNKS = _PER_W // _CHUNK

_NBINS = 128
_HC_WORDS = 2 * _NBINS * _L
_HB_WORDS = _NBINS * _L
_ROW = 3 * _NBINS


def _sc_hist(in_hbm, tg_hbm, out_hbm, buf_a, buf_b, hist_c, hist_b, out_row):
    wid = lax.axis_index("s") * _NC + lax.axis_index("c")
    base = wid * _PER_W
    lane = lax.broadcasted_iota(jnp.int32, (_L,), 0)
    ones = jnp.ones((_L,), jnp.int32)
    zeros = jnp.zeros((_L,), jnp.int32)

    def zero_c(j, carry):
        hist_c[pl.ds(j * _L, _L)] = zeros
        return carry

    lax.fori_loop(0, _HC_WORDS // _L, zero_c, 0)

    def zero_b(j, carry):
        hist_b[pl.ds(j * _L, _L)] = zeros
        return carry

    lax.fori_loop(0, _HB_WORDS // _L, zero_b, 0)

    def chunk_body(c, carry):
        off = base + c * _CHUNK
        pltpu.sync_copy(in_hbm.at[pl.ds(off, _CHUNK)], buf_a)
        pltpu.sync_copy(tg_hbm.at[pl.ds(off, _CHUNK)], buf_b)

        def inner(i, icarry):
            a = buf_a[pl.ds(i * _L, _L)].astype(jnp.int32)
            b = buf_b[pl.ds(i * _L, _L)].astype(jnp.int32)
            hit = jnp.where(a == b, _NBINS * _L, 0)
            idx_c = a * _L + lane + hit
            idx_b = b * _L + lane
            plsc.addupdate_scatter(hist_c, [idx_c], ones)
            plsc.addupdate_scatter(hist_b, [idx_b], ones)
            return icarry

        lax.fori_loop(0, _CHUNK // _L, inner, 0)
        return carry

    lax.fori_loop(0, _N_CHUNKS, chunk_body, 0)

    def red_c(g, carry):
        acc = zeros
        for l in range(_L):
            acc = acc + plsc.load_gather(hist_c, [g * (_L * _L) + lane * _L + l])
        out_row[pl.ds(g * _L, _L)] = acc
        return carry

    lax.fori_loop(0, 2 * _NBINS // _L, red_c, 0)

    def red_b(g, carry):
        acc = zeros
        for l in range(_L):
            acc = acc + plsc.load_gather(hist_b, [g * (_L * _L) + lane * _L + l])
        out_row[pl.ds(2 * _NBINS + g * _L, _L)] = acc
        return carry

    lax.fori_loop(0, _NBINS // _L, red_b, 0)

    pltpu.sync_copy(out_row, out_hbm.at[wid])


def _iou_epilogue(hist_ref, smooth_ref, out_ref):
    h = hist_ref[...].astype(jnp.float32)
    s = jnp.sum(h, axis=0, keepdims=True)
    lo = s[:, 0:_NBINS]
    hi = s[:, _NBINS : 2 * _NBINS]
    cb = s[:, 2 * _NBINS : 3 * _NBINS]
    ca = lo + hi
    union = ca + cb - hi
    valid = (union != 0.0).astype(jnp.float32)
    sm = smooth_ref[0, 0]
    iou = (hi + sm) / (union + sm)
    num = jnp.sum(valid, keepdims=True)
    tot = jnp.sum(iou * valid, keepdims=True)
    out_ref[...] = (tot / num).reshape(1, 1)


def kernel(inputs, targets, smooth=1):
    flat_in = inputs.reshape(_N_TOTAL)
    flat_tg = targets.reshape(_N_TOTAL)

    sc_call = pl.kernel(
        _sc_hist,
        out_type=jax.ShapeDtypeStruct((_NW, _ROW), jnp.int32),
        mesh=plsc.VectorSubcoreMesh(core_axis_name="c", subcore_axis_name="s"),
        compiler_params=pltpu.CompilerParams(needs_layout_passes=False),
        scratch_types=[
            pltpu.VMEM((_CHUNK,), jnp.float32),
            pltpu.VMEM((_CHUNK,), jnp.float32),
            pltpu.VMEM((_HC_WORDS,), jnp.int32),
            pltpu.VMEM((_HB_WORDS,), jnp.int32),
            pltpu.VMEM((_ROW,), jnp.int32),
        ],
    )
    hists = sc_call(flat_in, flat_tg)

    smooth_arr = jnp.float32(smooth).reshape(1, 1)
    out = pl.pallas_call(
        _iou_epilogue,
        out_shape=jax.ShapeDtypeStruct((1, 1), jnp.float32),
        in_specs=[
            pl.BlockSpec(memory_space=pltpu.VMEM),
            pl.BlockSpec(memory_space=pltpu.SMEM),
        ],
        out_specs=pl.BlockSpec(memory_space=pltpu.VMEM),
    )(hists, smooth_arr)
    return out[0, 0]


# speedup vs baseline: 39.2870x; 1.1366x over previous
import functools

import jax
import jax.numpy as jnp
from jax import lax
from jax.experimental import pallas as pl
from jax.experimental.pallas import tpu as pltpu
from jax.experimental.pallas import tpu_sc as plsc

_NC = 2
_NS = 16
_NW = _NC * _NS
_L = 16

_N_TOTAL = 64 * 512 * 512
_PER_W = _N_TOTAL // _NW
_CHUNK = 16384
_N_CHUNKS = _PER_W // _CHUNK
_UNROLL = 8

_NBINS = 128
_HC_WORDS = 2 * _NBINS * _L
_HB_WORDS = _NBINS * _L
_ROW = 3 * _NBINS


def _sc_hist(
    in_hbm,
    tg_hbm,
    out_hbm,
    buf_a0,
    buf_b0,
    buf_a1,
    buf_b1,
    sem0,
    sem1,
    hist_c,
    hist_b,
    out_row,
):
    wid = lax.axis_index("s") * _NC + lax.axis_index("c")
    base = wid * _PER_W
    lane = lax.broadcasted_iota(jnp.int32, (_L,), 0)
    ones = jnp.ones((_L,), jnp.int32)
    zeros = jnp.zeros((_L,), jnp.int32)

    def zero_c(j, carry):
        hist_c[pl.ds(j * _L, _L)] = zeros
        return carry

    lax.fori_loop(0, _HC_WORDS // _L, zero_c, 0)

    def zero_b(j, carry):
        hist_b[pl.ds(j * _L, _L)] = zeros
        return carry

    lax.fori_loop(0, _HB_WORDS // _L, zero_b, 0)

    bufs = ((buf_a0, buf_b0, sem0), (buf_a1, buf_b1, sem1))

    def start(c, slot):
        ba, bb, sem = bufs[slot]
        off = base + c * _CHUNK
        pltpu.async_copy(in_hbm.at[pl.ds(off, _CHUNK)], ba, sem)
        pltpu.async_copy(tg_hbm.at[pl.ds(off, _CHUNK)], bb, sem)

    def wait(slot):
        ba, bb, sem = bufs[slot]
        pltpu.make_async_copy(in_hbm.at[pl.ds(0, _CHUNK)], ba, sem).wait()
        pltpu.make_async_copy(tg_hbm.at[pl.ds(0, _CHUNK)], bb, sem).wait()

    def consume(slot):
        ba, bb, _ = bufs[slot]

        def inner(i, icarry):
            for u in range(_UNROLL):
                o = (i * _UNROLL + u) * _L
                a = ba[pl.ds(o, _L)].astype(jnp.int32)
                b = bb[pl.ds(o, _L)].astype(jnp.int32)
                hit = jnp.where(a == b, _NBINS * _L, 0)
                idx_c = a * _L + lane + hit
                idx_b = b * _L + lane
                plsc.addupdate_scatter(hist_c, [idx_c], ones)
                plsc.addupdate_scatter(hist_b, [idx_b], ones)
            return icarry

        lax.fori_loop(0, _CHUNK // (_L * _UNROLL), inner, 0)

    start(0, 0)

    def chunk_body(it, carry):
        c = 2 * it
        wait(0)

        @pl.when(c + 1 < _N_CHUNKS)
        def _():
            start(c + 1, 1)

        consume(0)
        wait(1)

        @pl.when(c + 2 < _N_CHUNKS)
        def _():
            start(c + 2, 0)

        consume(1)
        return carry

    lax.fori_loop(0, _N_CHUNKS // 2, chunk_body, 0)

    def red_c(g, carry):
        acc = zeros
        for l in range(_L):
            acc = acc + plsc.load_gather(hist_c, [g * (_L * _L) + lane * _L + l])
        out_row[pl.ds(g * _L, _L)] = acc
        return carry

    lax.fori_loop(0, 2 * _NBINS // _L, red_c, 0)

    def red_b(g, carry):
        acc = zeros
        for l in range(_L):
            acc = acc + plsc.load_gather(hist_b, [g * (_L * _L) + lane * _L + l])
        out_row[pl.ds(2 * _NBINS + g * _L, _L)] = acc
        return carry

    lax.fori_loop(0, _NBINS // _L, red_b, 0)

    pltpu.sync_copy(out_row, out_hbm.at[wid])


def _iou_epilogue(hist_ref, smooth_ref, out_ref):
    h = hist_ref[...].astype(jnp.float32)
    s = jnp.sum(h, axis=0, keepdims=True)
    lo = s[:, 0:_NBINS]
    hi = s[:, _NBINS : 2 * _NBINS]
    cb = s[:, 2 * _NBINS : 3 * _NBINS]
    ca = lo + hi
    union = ca + cb - hi
    valid = (union != 0.0).astype(jnp.float32)
    sm = smooth_ref[0, 0]
    iou = (hi + sm) / (union + sm)
    num = jnp.sum(valid, keepdims=True)
    tot = jnp.sum(iou * valid, keepdims=True)
    out_ref[...] = (tot / num).reshape(1, 1)


def kernel(inputs, targets, smooth=1):
    flat_in = inputs.reshape(_N_TOTAL)
    flat_tg = targets.reshape(_N_TOTAL)

    sc_call = pl.kernel(
        _sc_hist,
        out_type=jax.ShapeDtypeStruct((_NW, _ROW), jnp.int32),
        mesh=plsc.VectorSubcoreMesh(core_axis_name="c", subcore_axis_name="s"),
        compiler_params=pltpu.CompilerParams(needs_layout_passes=False),
        scratch_types=[
            pltpu.VMEM((_CHUNK,), jnp.float32),
            pltpu.VMEM((_CHUNK,), jnp.float32),
            pltpu.VMEM((_CHUNK,), jnp.float32),
            pltpu.VMEM((_CHUNK,), jnp.float32),
            pltpu.SemaphoreType.DMA,
            pltpu.SemaphoreType.DMA,
            pltpu.VMEM((_HC_WORDS,), jnp.int32),
            pltpu.VMEM((_HB_WORDS,), jnp.int32),
            pltpu.VMEM((_ROW,), jnp.int32),
        ],
    )
    hists = sc_call(flat_in, flat_tg)

    smooth_arr = jnp.float32(smooth).reshape(1, 1)
    out = pl.pallas_call(
        _iou_epilogue,
        out_shape=jax.ShapeDtypeStruct((1, 1), jnp.float32),
        in_specs=[
            pl.BlockSpec(memory_space=pltpu.VMEM),
            pl.BlockSpec(memory_space=pltpu.SMEM),
        ],
        out_specs=pl.BlockSpec(memory_space=pltpu.VMEM),
    )(hists, smooth_arr)
    return out[0, 0]


# speedup vs baseline: 92.5481x; 2.3557x over previous
import functools

import jax
import jax.numpy as jnp
from jax import lax
from jax.experimental import pallas as pl
from jax.experimental.pallas import tpu as pltpu
from jax.experimental.pallas import tpu_sc as plsc

_NC = 2
_NS = 16
_NW = _NC * _NS
_L = 16

_N_TOTAL = 64 * 512 * 512
_PER_W = _N_TOTAL // _NW
_CHUNK = 16384
_N_CHUNKS = _PER_W // _CHUNK
_UNROLL = 8

_NBINS = 128
_HC_WORDS = 2 * _NBINS * _L
_HB_WORDS = _NBINS * _L
_ROW = 3 * _NBINS


def _sc_hist(
    in_hbm,
    tg_hbm,
    out_hbm,
    buf_a0,
    buf_b0,
    buf_a1,
    buf_b1,
    sem0,
    sem1,
    hist_c,
    hist_b,
    out_row,
):
    wid = lax.axis_index("s") * _NC + lax.axis_index("c")
    base = wid * _PER_W
    lane = lax.broadcasted_iota(jnp.int32, (_L,), 0)
    ones = jnp.ones((_L,), jnp.int32)
    zeros = jnp.zeros((_L,), jnp.int32)

    def zero_c(j, carry):
        hist_c[pl.ds(j * _L, _L)] = zeros
        return carry

    lax.fori_loop(0, _HC_WORDS // _L, zero_c, 0)

    def zero_b(j, carry):
        hist_b[pl.ds(j * _L, _L)] = zeros
        return carry

    lax.fori_loop(0, _HB_WORDS // _L, zero_b, 0)

    bufs = ((buf_a0, buf_b0, sem0), (buf_a1, buf_b1, sem1))

    def start(c, slot):
        ba, bb, sem = bufs[slot]
        off = base + c * _CHUNK
        pltpu.async_copy(in_hbm.at[pl.ds(off, _CHUNK)], ba, sem)
        pltpu.async_copy(tg_hbm.at[pl.ds(off, _CHUNK)], bb, sem)

    def wait(slot):
        ba, bb, sem = bufs[slot]
        pltpu.make_async_copy(in_hbm.at[pl.ds(0, _CHUNK)], ba, sem).wait()
        pltpu.make_async_copy(tg_hbm.at[pl.ds(0, _CHUNK)], bb, sem).wait()

    def consume(slot):
        ba, bb, _ = bufs[slot]

        @plsc.parallel_loop(0, _CHUNK // _L, unroll=_UNROLL)
        def _inner(i):
            o = i * _L
            a = ba[pl.ds(o, _L)].astype(jnp.int32)
            b = bb[pl.ds(o, _L)].astype(jnp.int32)
            hit = jnp.where(a == b, _NBINS * _L, 0)
            idx_c = jnp.bitwise_or(a * _L + hit, lane)
            idx_b = jnp.bitwise_or(b * _L, lane)
            plsc.addupdate_scatter(hist_c, [idx_c], ones)
            plsc.addupdate_scatter(hist_b, [idx_b], ones)

    start(0, 0)

    def chunk_body(it, carry):
        c = 2 * it
        wait(0)

        @pl.when(c + 1 < _N_CHUNKS)
        def _():
            start(c + 1, 1)

        consume(0)
        wait(1)

        @pl.when(c + 2 < _N_CHUNKS)
        def _():
            start(c + 2, 0)

        consume(1)
        return carry

    lax.fori_loop(0, _N_CHUNKS // 2, chunk_body, 0)

    def red_c(g, carry):
        acc = zeros
        for l in range(_L):
            acc = acc + plsc.load_gather(hist_c, [g * (_L * _L) + lane * _L + l])
        out_row[pl.ds(g * _L, _L)] = acc
        return carry

    lax.fori_loop(0, 2 * _NBINS // _L, red_c, 0)

    def red_b(g, carry):
        acc = zeros
        for l in range(_L):
            acc = acc + plsc.load_gather(hist_b, [g * (_L * _L) + lane * _L + l])
        out_row[pl.ds(2 * _NBINS + g * _L, _L)] = acc
        return carry

    lax.fori_loop(0, _NBINS // _L, red_b, 0)

    pltpu.sync_copy(out_row, out_hbm.at[wid])


def _iou_epilogue(hist_ref, smooth_ref, out_ref):
    h = hist_ref[...].astype(jnp.float32)
    s = jnp.sum(h, axis=0, keepdims=True)
    lo = s[:, 0:_NBINS]
    hi = s[:, _NBINS : 2 * _NBINS]
    cb = s[:, 2 * _NBINS : 3 * _NBINS]
    ca = lo + hi
    union = ca + cb - hi
    valid = (union != 0.0).astype(jnp.float32)
    sm = smooth_ref[0, 0]
    iou = (hi + sm) / (union + sm)
    num = jnp.sum(valid, keepdims=True)
    tot = jnp.sum(iou * valid, keepdims=True)
    out_ref[...] = (tot / num).reshape(1, 1)


def kernel(inputs, targets, smooth=1):
    flat_in = inputs.reshape(_N_TOTAL)
    flat_tg = targets.reshape(_N_TOTAL)

    sc_call = pl.kernel(
        _sc_hist,
        out_type=jax.ShapeDtypeStruct((_NW, _ROW), jnp.int32),
        mesh=plsc.VectorSubcoreMesh(core_axis_name="c", subcore_axis_name="s"),
        compiler_params=pltpu.CompilerParams(needs_layout_passes=False),
        scratch_types=[
            pltpu.VMEM((_CHUNK,), jnp.float32),
            pltpu.VMEM((_CHUNK,), jnp.float32),
            pltpu.VMEM((_CHUNK,), jnp.float32),
            pltpu.VMEM((_CHUNK,), jnp.float32),
            pltpu.SemaphoreType.DMA,
            pltpu.SemaphoreType.DMA,
            pltpu.VMEM((_HC_WORDS,), jnp.int32),
            pltpu.VMEM((_HB_WORDS,), jnp.int32),
            pltpu.VMEM((_ROW,), jnp.int32),
        ],
    )
    hists = sc_call(flat_in, flat_tg)

    smooth_arr = jnp.float32(smooth).reshape(1, 1)
    out = pl.pallas_call(
        _iou_epilogue,
        out_shape=jax.ShapeDtypeStruct((1, 1), jnp.float32),
        in_specs=[
            pl.BlockSpec(memory_space=pltpu.VMEM),
            pl.BlockSpec(memory_space=pltpu.SMEM),
        ],
        out_specs=pl.BlockSpec(memory_space=pltpu.VMEM),
    )(hists, smooth_arr)
    return out[0, 0]


# speedup vs baseline: 171.6092x; 1.8543x over previous
import functools

import jax
import jax.numpy as jnp
from jax import lax
from jax.experimental import pallas as pl
from jax.experimental.pallas import tpu as pltpu
from jax.experimental.pallas import tpu_sc as plsc

_NC = 2
_NS = 16
_NW = _NC * _NS
_L = 16

_N_TOTAL = 64 * 512 * 512
_IMGS = 64
_ROWS = 512
_COLS = 512
_IMG_PER_W = _IMGS // _NW
_R_CHUNK = 32
_CHUNK = _R_CHUNK * _COLS
_N_CHUNKS = _IMG_PER_W * _ROWS // _R_CHUNK
_CPI = _ROWS // _R_CHUNK
_UNROLL = 8

_NBINS = 128
_HC_WORDS = 2 * _NBINS * _L
_HB_WORDS = _NBINS * _L
_ROW = 3 * _NBINS


def _sc_hist(
    in_hbm,
    tg_hbm,
    out_hbm,
    buf_a0,
    buf_b0,
    buf_a1,
    buf_b1,
    sem0,
    sem1,
    hist_c,
    hist_b,
    out_row,
):
    wid = lax.axis_index("s") * _NC + lax.axis_index("c")
    lane = lax.broadcasted_iota(jnp.int32, (_L,), 0)
    ones = jnp.ones((_L,), jnp.int32)
    zeros = jnp.zeros((_L,), jnp.int32)

    def zero_c(j, carry):
        hist_c[pl.ds(j * _L, _L)] = zeros
        return carry

    lax.fori_loop(0, _HC_WORDS // _L, zero_c, 0)

    def zero_b(j, carry):
        hist_b[pl.ds(j * _L, _L)] = zeros
        return carry

    lax.fori_loop(0, _HB_WORDS // _L, zero_b, 0)

    bufs = ((buf_a0, buf_b0, sem0), (buf_a1, buf_b1, sem1))

    def start(c, slot):
        ba, bb, sem = bufs[slot]
        img = _IMG_PER_W * wid + c // _CPI
        r0 = (c % _CPI) * _R_CHUNK
        pltpu.async_copy(in_hbm.at[img, pl.ds(r0, _R_CHUNK), :], ba, sem)
        pltpu.async_copy(tg_hbm.at[img, pl.ds(r0, _R_CHUNK), :], bb, sem)

    def wait(slot):
        ba, bb, sem = bufs[slot]
        pltpu.make_async_copy(in_hbm.at[0, pl.ds(0, _R_CHUNK), :], ba, sem).wait()
        pltpu.make_async_copy(tg_hbm.at[0, pl.ds(0, _R_CHUNK), :], bb, sem).wait()

    def consume(slot):
        ba, bb, _ = bufs[slot]

        @plsc.parallel_loop(0, _CHUNK // _L, unroll=_UNROLL)
        def _inner(i):
            r = i // (_COLS // _L)
            o = (i % (_COLS // _L)) * _L
            a = ba[r, pl.ds(o, _L)].astype(jnp.int32)
            b = bb[r, pl.ds(o, _L)].astype(jnp.int32)
            hit = jnp.where(a == b, _NBINS * _L, 0)
            idx_c = jnp.bitwise_or(a * _L + hit, lane)
            idx_b = jnp.bitwise_or(b * _L, lane)
            plsc.addupdate_scatter(hist_c, [idx_c], ones)
            plsc.addupdate_scatter(hist_b, [idx_b], ones)

    start(0, 0)

    def chunk_body(it, carry):
        c = 2 * it
        wait(0)

        @pl.when(c + 1 < _N_CHUNKS)
        def _():
            start(c + 1, 1)

        consume(0)
        wait(1)

        @pl.when(c + 2 < _N_CHUNKS)
        def _():
            start(c + 2, 0)

        consume(1)
        return carry

    lax.fori_loop(0, _N_CHUNKS // 2, chunk_body, 0)

    def red_c(g, carry):
        acc = zeros
        for l in range(_L):
            acc = acc + plsc.load_gather(hist_c, [g * (_L * _L) + lane * _L + l])
        out_row[pl.ds(g * _L, _L)] = acc
        return carry

    lax.fori_loop(0, 2 * _NBINS // _L, red_c, 0)

    def red_b(g, carry):
        acc = zeros
        for l in range(_L):
            acc = acc + plsc.load_gather(hist_b, [g * (_L * _L) + lane * _L + l])
        out_row[pl.ds(2 * _NBINS + g * _L, _L)] = acc
        return carry

    lax.fori_loop(0, _NBINS // _L, red_b, 0)

    pltpu.sync_copy(out_row, out_hbm.at[wid])


def _iou_epilogue(hist_ref, smooth_ref, out_ref):
    h = hist_ref[...].astype(jnp.float32)
    s = jnp.sum(h, axis=0, keepdims=True)
    lo = s[:, 0:_NBINS]
    hi = s[:, _NBINS : 2 * _NBINS]
    cb = s[:, 2 * _NBINS : 3 * _NBINS]
    ca = lo + hi
    union = ca + cb - hi
    valid = (union != 0.0).astype(jnp.float32)
    sm = smooth_ref[0, 0]
    iou = (hi + sm) / (union + sm)
    num = jnp.sum(valid, keepdims=True)
    tot = jnp.sum(iou * valid, keepdims=True)
    out_ref[...] = (tot / num).reshape(1, 1)


def kernel(inputs, targets, smooth=1):
    sc_call = pl.kernel(
        _sc_hist,
        out_type=jax.ShapeDtypeStruct((_NW, _ROW), jnp.int32),
        mesh=plsc.VectorSubcoreMesh(core_axis_name="c", subcore_axis_name="s"),
        compiler_params=pltpu.CompilerParams(
            needs_layout_passes=False, use_tc_tiling_on_sc=True
        ),
        scratch_types=[
            pltpu.VMEM((_R_CHUNK, _COLS), jnp.float32),
            pltpu.VMEM((_R_CHUNK, _COLS), jnp.float32),
            pltpu.VMEM((_R_CHUNK, _COLS), jnp.float32),
            pltpu.VMEM((_R_CHUNK, _COLS), jnp.float32),
            pltpu.SemaphoreType.DMA,
            pltpu.SemaphoreType.DMA,
            pltpu.VMEM((_HC_WORDS,), jnp.int32),
            pltpu.VMEM((_HB_WORDS,), jnp.int32),
            pltpu.VMEM((_ROW,), jnp.int32),
        ],
    )
    hists = sc_call(inputs, targets)

    smooth_arr = jnp.float32(smooth).reshape(1, 1)
    out = pl.pallas_call(
        _iou_epilogue,
        out_shape=jax.ShapeDtypeStruct((1, 1), jnp.float32),
        in_specs=[
            pl.BlockSpec(memory_space=pltpu.VMEM),
            pl.BlockSpec(memory_space=pltpu.SMEM),
        ],
        out_specs=pl.BlockSpec(memory_space=pltpu.VMEM),
    )(hists, smooth_arr)
    return out[0, 0]


# speedup vs baseline: 192.5226x; 1.1219x over previous
import functools

import jax
import jax.numpy as jnp
from jax import lax
from jax.experimental import pallas as pl
from jax.experimental.pallas import tpu as pltpu
from jax.experimental.pallas import tpu_sc as plsc

_NC = 2
_NS = 16
_NW = _NC * _NS
_L = 16

_N_TOTAL = 64 * 512 * 512
_IMGS = 64
_ROWS = 512
_COLS = 512
_IMG_PER_W = _IMGS // _NW
_R_CHUNK = 32
_CHUNK = _R_CHUNK * _COLS
_N_CHUNKS = _IMG_PER_W * _ROWS // _R_CHUNK
_CPI = _ROWS // _R_CHUNK
_UNROLL = 8

_NBINS = 128
_HA_WORDS = _NBINS * _L
_HB_WORDS = _NBINS * _L
_ROW = 3 * _NBINS


def _sc_hist(
    in_hbm,
    tg_hbm,
    out_hbm,
    buf_a0,
    buf_b0,
    buf_a1,
    buf_b1,
    sem0,
    sem1,
    hist_a,
    hist_b,
    out_row,
):
    wid = lax.axis_index("s") * _NC + lax.axis_index("c")
    lane = lax.broadcasted_iota(jnp.int32, (_L,), 0)
    ones = jnp.ones((_L,), jnp.int32)
    zeros = jnp.zeros((_L,), jnp.int32)

    def zero_a(j, carry):
        hist_a[pl.ds(j * _L, _L)] = zeros
        return carry

    lax.fori_loop(0, _HA_WORDS // _L, zero_a, 0)

    def zero_b(j, carry):
        hist_b[pl.ds(j * _L, _L)] = zeros
        return carry

    lax.fori_loop(0, _HB_WORDS // _L, zero_b, 0)

    bufs = ((buf_a0, buf_b0, sem0), (buf_a1, buf_b1, sem1))

    def start(c, slot):
        ba, bb, sem = bufs[slot]
        img = _IMG_PER_W * wid + c // _CPI
        r0 = (c % _CPI) * _R_CHUNK
        pltpu.async_copy(in_hbm.at[img, pl.ds(r0, _R_CHUNK), :], ba, sem)
        pltpu.async_copy(tg_hbm.at[img, pl.ds(r0, _R_CHUNK), :], bb, sem)

    def wait(slot):
        ba, bb, sem = bufs[slot]
        pltpu.make_async_copy(in_hbm.at[0, pl.ds(0, _R_CHUNK), :], ba, sem).wait()
        pltpu.make_async_copy(tg_hbm.at[0, pl.ds(0, _R_CHUNK), :], bb, sem).wait()

    def consume(slot):
        ba, bb, _ = bufs[slot]

        @plsc.parallel_loop(0, _CHUNK // _L, unroll=_UNROLL)
        def _inner(i):
            r = i // (_COLS // _L)
            o = (i % (_COLS // _L)) * _L
            af = ba[r, pl.ds(o, _L)]
            bf = bb[r, pl.ds(o, _L)]
            val = jnp.where(af == bf, 65537, 1)
            a = af.astype(jnp.int32)
            b = bf.astype(jnp.int32)
            idx_a = jnp.bitwise_or(a * _L, lane)
            idx_b = jnp.bitwise_or(b * _L, lane)
            plsc.addupdate_scatter(hist_a, [idx_a], val)
            plsc.addupdate_scatter(hist_b, [idx_b], ones)

    start(0, 0)

    def chunk_body(it, carry):
        c = 2 * it
        wait(0)

        @pl.when(c + 1 < _N_CHUNKS)
        def _():
            start(c + 1, 1)

        consume(0)
        wait(1)

        @pl.when(c + 2 < _N_CHUNKS)
        def _():
            start(c + 2, 0)

        consume(1)
        return carry

    lax.fori_loop(0, _N_CHUNKS // 2, chunk_body, 0)

    def red_a(g, carry):
        acc_lo = zeros
        acc_hi = zeros
        for l in range(_L):
            v = plsc.load_gather(hist_a, [g * (_L * _L) + lane * _L + l])
            acc_lo = acc_lo + jnp.bitwise_and(v, 0xFFFF)
            acc_hi = acc_hi + lax.shift_right_logical(v, 16)
        out_row[pl.ds(g * _L, _L)] = acc_lo
        out_row[pl.ds(_NBINS + g * _L, _L)] = acc_hi
        return carry

    lax.fori_loop(0, _NBINS // _L, red_a, 0)

    def red_b(g, carry):
        acc = zeros
        for l in range(_L):
            acc = acc + plsc.load_gather(hist_b, [g * (_L * _L) + lane * _L + l])
        out_row[pl.ds(2 * _NBINS + g * _L, _L)] = acc
        return carry

    lax.fori_loop(0, _NBINS // _L, red_b, 0)

    pltpu.sync_copy(out_row, out_hbm.at[wid])


def _iou_epilogue(hist_ref, smooth_ref, out_ref):
    h = hist_ref[...].astype(jnp.float32)
    s = jnp.sum(h, axis=0, keepdims=True)
    ca = s[:, 0:_NBINS]
    hi = s[:, _NBINS : 2 * _NBINS]
    cb = s[:, 2 * _NBINS : 3 * _NBINS]
    union = ca + cb - hi
    valid = (union != 0.0).astype(jnp.float32)
    sm = smooth_ref[0, 0]
    iou = (hi + sm) / (union + sm)
    num = jnp.sum(valid, keepdims=True)
    tot = jnp.sum(iou * valid, keepdims=True)
    out_ref[...] = (tot / num).reshape(1, 1)


def kernel(inputs, targets, smooth=1):
    sc_call = pl.kernel(
        _sc_hist,
        out_type=jax.ShapeDtypeStruct((_NW, _ROW), jnp.int32),
        mesh=plsc.VectorSubcoreMesh(core_axis_name="c", subcore_axis_name="s"),
        compiler_params=pltpu.CompilerParams(
            needs_layout_passes=False, use_tc_tiling_on_sc=True
        ),
        scratch_types=[
            pltpu.VMEM((_R_CHUNK, _COLS), jnp.float32),
            pltpu.VMEM((_R_CHUNK, _COLS), jnp.float32),
            pltpu.VMEM((_R_CHUNK, _COLS), jnp.float32),
            pltpu.VMEM((_R_CHUNK, _COLS), jnp.float32),
            pltpu.SemaphoreType.DMA,
            pltpu.SemaphoreType.DMA,
            pltpu.VMEM((_HA_WORDS,), jnp.int32),
            pltpu.VMEM((_HB_WORDS,), jnp.int32),
            pltpu.VMEM((_ROW,), jnp.int32),
        ],
    )
    hists = sc_call(inputs, targets)

    smooth_arr = jnp.float32(smooth).reshape(1, 1)
    out = pl.pallas_call(
        _iou_epilogue,
        out_shape=jax.ShapeDtypeStruct((1, 1), jnp.float32),
        in_specs=[
            pl.BlockSpec(memory_space=pltpu.VMEM),
            pl.BlockSpec(memory_space=pltpu.SMEM),
        ],
        out_specs=pl.BlockSpec(memory_space=pltpu.VMEM),
    )(hists, smooth_arr)
    return out[0, 0]
